# baseline (speedup 1.0000x reference)
import functools
import math

import jax
import jax.numpy as jnp
from jax.experimental import pallas as pl
from jax.experimental.pallas import tpu as pltpu
from jax.experimental.pallas import tpu_sc as plsc

_B = 4
_L = 1024
_W = 32
_INC = 768
_OUTC = 512
_NS = 8
_CH = 192
_H = 8
_DH = 64
_T = 77
_E = 4


def _gelu_exact(x):
    return 0.5 * x * (1.0 + jax.lax.erf(x * (1.0 / math.sqrt(2.0))))


def _body(x0_ref, x1_ref, x2_ref, x3_ref, x4_ref, x5_ref, x6_ref, x7_ref, tf_ref, dcw_ref, dcb_ref,
          wtaps_ref, cvb_ref, proj_ref, n1w_ref, n1b_ref, n2w_ref, n2b_ref,
          wqT_ref, wk_ref, wv_ref, woT_ref, wob_ref, gw_ref, gb_ref,
          ew1_ref, eb1_ref, ew2_ref, eb2_ref, out_ref, eo_ref, f1_s,
          pooled_s):
    b = pl.program_id(0)
    g = pl.program_id(1)

    def stage1(base):
        acc = dcb_ref[...] * jnp.ones((1, _L), jnp.float32)
        for j, xr in enumerate((x0_ref, x1_ref, x2_ref, x3_ref, x4_ref, x5_ref, x6_ref, x7_ref)):
            xj = xr[0, 0].astype(jnp.bfloat16)
            wj = dcw_ref[:, base + j * _CH:base + (j + 1) * _CH].astype(
                jnp.bfloat16)
            acc = acc + jnp.dot(wj, xj, preferred_element_type=jnp.float32)
        return acc

    @pl.when(g == 0)
    def _():
        f1_s[...] = stage1(0)

    @pl.when(g == 1)
    def _():
        f1_s[...] += stage1(_NS * _CH)

        f1 = f1_s[...].astype(jnp.bfloat16)

        lane = jax.lax.broadcasted_iota(jnp.int32, (1, _L), 1)
        p_ = lane // _W
        q_ = lane % _W
        acc = jnp.zeros((_OUTC, _L), jnp.float32)
        for grp in range(3):
            parts = []
            for t in range(3 * grp, 3 * grp + 3):
                a, c = t // 3, t % 3
                s = (a - 1) * _W + (c - 1)
                shifted = jnp.roll(f1, -s, axis=1) if s != 0 else f1
                valid = ((q_ + (c - 1) >= 0) & (q_ + (c - 1) < _W)
                         & (p_ + (a - 1) >= 0) & (p_ + (a - 1) < _W))
                parts.append(jnp.where(valid, shifted, jnp.bfloat16(0.0)))
            gmat = jnp.concatenate(parts, axis=0)
            wslice = wtaps_ref[:, grp * 3 * _INC:(grp + 1) * 3 * _INC]
            acc = acc + jnp.dot(wslice, gmat,
                                preferred_element_type=jnp.float32)
        f2 = jnp.maximum(acc + cvb_ref[...], 0.0)

        m = jnp.mean(f2, axis=0, keepdims=True)
        v = jnp.mean((f2 - m) ** 2, axis=0, keepdims=True)
        f_ln = (f2 - m) / jnp.sqrt(v + 1e-5) * n1w_ref[...] + n1b_ref[...]

        tf = tf_ref[0]
        ctx = jnp.dot(tf, proj_ref[...], preferred_element_type=jnp.float32)
        cm = jnp.mean(ctx, axis=1, keepdims=True)
        cv = jnp.mean((ctx - cm) ** 2, axis=1, keepdims=True)
        ctxn = (ctx - cm) / jnp.sqrt(cv + 1e-5) * n2w_ref[...] + n2b_ref[...]

        krm = jnp.dot(ctxn, wk_ref[...], preferred_element_type=jnp.float32)
        vrm = jnp.dot(ctxn, wv_ref[...], preferred_element_type=jnp.float32)
        qcm = jnp.dot(wqT_ref[...], f_ln, preferred_element_type=jnp.float32)

        scale = 1.0 / math.sqrt(_DH)
        outs = []
        for h in range(_H):
            kh = krm[:, h * _DH:(h + 1) * _DH]
            qh = qcm[h * _DH:(h + 1) * _DH, :]
            simT = jnp.dot(kh, qh, preferred_element_type=jnp.float32) * scale
            mx = jnp.max(simT, axis=0, keepdims=True)
            ex = jnp.exp(simT - mx)
            attnT = ex / jnp.sum(ex, axis=0, keepdims=True)
            vh = vrm[:, h * _DH:(h + 1) * _DH]
            oh = jax.lax.dot_general(vh, attnT, (((0,), (0,)), ((), ())),
                                     preferred_element_type=jnp.float32)
            outs.append(oh)
        ocm = jnp.concatenate(outs, axis=0)
        o2 = jnp.dot(woT_ref[...], ocm,
                     preferred_element_type=jnp.float32) + wob_ref[...]
        fsum = f_ln + o2

        ones_row = jnp.ones((1, _L), jnp.float32)
        prow = jax.lax.dot_general(ones_row, fsum, (((1,), (1,)), ((), ())),
                                   preferred_element_type=jnp.float32) / _L
        pooled_s[pl.ds(b, 1), :] = prow

    @pl.when((b == _B - 1) & (g == 1))
    def _():
        pooled = pooled_s[...]
        glog = jnp.dot(pooled, gw_ref[...],
                       preferred_element_type=jnp.float32) + gb_ref[...]

        eos = []
        for e in range(_E):
            hh = jnp.dot(pooled, ew1_ref[e],
                         preferred_element_type=jnp.float32) + eb1_ref[e]
            hh = _gelu_exact(hh)
            eo_e = jnp.dot(hh, ew2_ref[e],
                           preferred_element_type=jnp.float32) + eb2_ref[e]
            eos.append(eo_e)
        out_ref[...] = glog
        eo_ref[...] = jnp.concatenate(eos, axis=1)


def _sc_route(glog16, eo16):
    mesh = plsc.VectorSubcoreMesh(core_axis_name="c", subcore_axis_name="s")

    @functools.partial(
        pl.kernel, mesh=mesh,
        out_type=jax.ShapeDtypeStruct((16,), jnp.float32),
        scratch_types=[pltpu.VMEM((16,), jnp.float32),
                       pltpu.VMEM((16,), jnp.float32),
                       pltpu.VMEM((16,), jnp.float32)],
    )
    def _route(glog_hbm, eo_hbm, out_hbm, g_v, e_v, o_v):
        cid = jax.lax.axis_index("c")
        sid = jax.lax.axis_index("s")

        @pl.when((cid == 0) & (sid == 0))
        def _():
            pltpu.sync_copy(glog_hbm, g_v)
            pltpu.sync_copy(eo_hbm, e_v)
            g = g_v[...]
            eo = e_v[...]
            idx = jax.lax.broadcasted_iota(jnp.int32, (16,), 0)

            def bfly(x, op):
                x = op(x, jnp.take(x, idx ^ 1))
                return op(x, jnp.take(x, idx ^ 2))

            m = bfly(g, jnp.maximum)
            ex = jnp.exp(g - m)
            gs = ex / bfly(ex, jnp.add)
            gmin = bfly(gs, jnp.minimum)
            sel = jnp.where(gs <= gmin, idx, -1)
            excl = bfly(sel, jnp.maximum)
            t = jnp.where(idx != excl, gs * eo, jnp.float32(0.0))
            p = bfly(t, jnp.add)
            o_v[...] = jnp.take(p, (idx * 4) & 15)
            pltpu.sync_copy(o_v, out_hbm)

    return _route(glog16, eo16)


def kernel(x, text_features, dc_w, dc_b, conv_w, conv_b, proj, norm1_w,
           norm1_b, norm2_w, norm2_b, wq, wk, wv, wo, wo_b, gate_w, gate_b,
           e_w1, e_b1, e_w2, e_b2):
    B = x.shape[0]
    xr = x.reshape(B, 2 * _NS, _CH, _L)
    dcw = dc_w.reshape(_INC, _INC * 4)
    wtaps = conv_w.transpose(0, 2, 3, 1).reshape(_OUTC, 9 * _INC).astype(
        jnp.bfloat16)

    grid = (B, 2)

    def const(*block):
        return pl.BlockSpec(block, lambda b, g: tuple(0 for _ in block))

    def xspec(j):
        return pl.BlockSpec((1, 1, _CH, _L),
                            lambda b, g, j=j: (b, g * _NS + j, 0, 0))

    in_specs = [
        xspec(0), xspec(1), xspec(2), xspec(3),
        xspec(4), xspec(5), xspec(6), xspec(7),
        pl.BlockSpec((1, _T, _INC), lambda b, g: (b, 0, 0)),
        const(_INC, _INC * 4),
        const(_INC, 1),
        const(_OUTC, 9 * _INC),
        const(_OUTC, 1),
        const(_INC, _OUTC),
        const(_OUTC, 1), const(_OUTC, 1),
        const(1, _OUTC), const(1, _OUTC),
        const(_OUTC, _OUTC),
        const(_OUTC, _OUTC),
        const(_OUTC, _OUTC),
        const(_OUTC, _OUTC),
        const(_OUTC, 1),
        const(_OUTC, _E),
        const(1, _E),
        const(_E, _OUTC, _OUTC),
        const(_E, 1, _OUTC),
        const(_E, _OUTC, 1),
        const(_E, 1, 1),
    ]

    glog, eo = pl.pallas_call(
        _body,
        grid=grid,
        in_specs=in_specs,
        out_specs=[pl.BlockSpec((_B, _E), lambda b, g: (0, 0)),
                   pl.BlockSpec((_B, _E), lambda b, g: (0, 0))],
        out_shape=[jax.ShapeDtypeStruct((_B, _E), jnp.float32),
                   jax.ShapeDtypeStruct((_B, _E), jnp.float32)],
        scratch_shapes=[
            pltpu.VMEM((_INC, _L), jnp.float32),
            pltpu.VMEM((_B, _OUTC), jnp.float32),
        ],
    )(xr, xr, xr, xr, xr, xr, xr, xr, text_features, dcw, dc_b.reshape(_INC, 1), wtaps,
      conv_b.reshape(_OUTC, 1), proj, norm1_w.reshape(_OUTC, 1),
      norm1_b.reshape(_OUTC, 1), norm2_w.reshape(1, _OUTC),
      norm2_b.reshape(1, _OUTC), wq.T, wk, wv, wo.T, wo_b.reshape(_OUTC, 1),
      gate_w, gate_b.reshape(1, _E), e_w1, e_b1.reshape(_E, 1, _OUTC),
      e_w2, e_b2.reshape(_E, 1, 1))

    pred16 = _sc_route(glog.reshape(_B * _E), eo.reshape(_B * _E))
    return pred16[:_B].reshape(_B, 1)


# speedup vs baseline: 1.2755x; 1.2755x over previous
import functools
import math

import jax
import jax.numpy as jnp
from jax.experimental import pallas as pl
from jax.experimental.pallas import tpu as pltpu
from jax.experimental.pallas import tpu_sc as plsc

_B = 4
_L = 1024
_W = 32
_INC = 768
_OUTC = 512
_NS = 4
_CH = 384
_H = 8
_DH = 64
_T = 77
_E = 4


def _gelu_exact(x):
    return 0.5 * x * (1.0 + jax.lax.erf(x * (1.0 / math.sqrt(2.0))))


def _body(x0_ref, x1_ref, x2_ref, x3_ref, tf_ref, dcw_ref, dcb_ref,
          wtaps_ref, cvb_ref, proj_ref, n1w_ref, n1b_ref, n2w_ref, n2b_ref,
          wqT_ref, wk_ref, wv_ref, woT_ref, wob_ref, gw_ref, gb_ref,
          ew1_ref, eb1_ref, ew2_ref, eb2_ref, out_ref, eo_ref, f1_s,
          pooled_s):
    b = pl.program_id(0)
    g = pl.program_id(1)

    def stage1(base):
        acc = dcb_ref[...] * jnp.ones((1, _L), jnp.float32)
        for j, xr in enumerate((x0_ref, x1_ref, x2_ref, x3_ref)):
            xj = xr[0, 0].astype(jnp.bfloat16)
            wj = dcw_ref[:, base + j * _CH:base + (j + 1) * _CH].astype(
                jnp.bfloat16)
            acc = acc + jnp.dot(wj, xj, preferred_element_type=jnp.float32)
        return acc

    @pl.when((b < _B) & (g == 0))
    def _():
        f1_s[b % 2] = stage1(0)

    @pl.when((b < _B) & (g == 1))
    def _():
        f1_s[b % 2] += stage1(_NS * _CH)

    @pl.when((b > 0) & (g == 0))
    def _():
        f1 = f1_s[1 - b % 2].astype(jnp.bfloat16)

        lane = jax.lax.broadcasted_iota(jnp.int32, (1, _L), 1)
        p_ = lane // _W
        q_ = lane % _W
        acc = jnp.zeros((_OUTC, _L), jnp.float32)
        for grp in range(3):
            parts = []
            for t in range(3 * grp, 3 * grp + 3):
                a, c = t // 3, t % 3
                s = (a - 1) * _W + (c - 1)
                shifted = jnp.roll(f1, -s, axis=1) if s != 0 else f1
                valid = ((q_ + (c - 1) >= 0) & (q_ + (c - 1) < _W)
                         & (p_ + (a - 1) >= 0) & (p_ + (a - 1) < _W))
                parts.append(jnp.where(valid, shifted, jnp.bfloat16(0.0)))
            gmat = jnp.concatenate(parts, axis=0)
            wslice = wtaps_ref[:, grp * 3 * _INC:(grp + 1) * 3 * _INC]
            acc = acc + jnp.dot(wslice, gmat,
                                preferred_element_type=jnp.float32)
        f2 = jnp.maximum(acc + cvb_ref[...], 0.0)

        m = jnp.mean(f2, axis=0, keepdims=True)
        v = jnp.mean((f2 - m) ** 2, axis=0, keepdims=True)
        f_ln = (f2 - m) / jnp.sqrt(v + 1e-5) * n1w_ref[...] + n1b_ref[...]

        tf = tf_ref[0]
        ctx = jnp.dot(tf, proj_ref[...], preferred_element_type=jnp.float32)
        cm = jnp.mean(ctx, axis=1, keepdims=True)
        cv = jnp.mean((ctx - cm) ** 2, axis=1, keepdims=True)
        ctxn = (ctx - cm) / jnp.sqrt(cv + 1e-5) * n2w_ref[...] + n2b_ref[...]

        krm = jnp.dot(ctxn, wk_ref[...], preferred_element_type=jnp.float32)
        vrm = jnp.dot(ctxn, wv_ref[...], preferred_element_type=jnp.float32)
        qcm = jnp.dot(wqT_ref[...], f_ln, preferred_element_type=jnp.float32)

        scale = 1.0 / math.sqrt(_DH)
        outs = []
        for h in range(_H):
            kh = krm[:, h * _DH:(h + 1) * _DH]
            qh = qcm[h * _DH:(h + 1) * _DH, :]
            simT = jnp.dot(kh, qh, preferred_element_type=jnp.float32) * scale
            mx = jnp.max(simT, axis=0, keepdims=True)
            ex = jnp.exp(simT - mx)
            attnT = ex / jnp.sum(ex, axis=0, keepdims=True)
            vh = vrm[:, h * _DH:(h + 1) * _DH]
            oh = jax.lax.dot_general(vh, attnT, (((0,), (0,)), ((), ())),
                                     preferred_element_type=jnp.float32)
            outs.append(oh)
        ocm = jnp.concatenate(outs, axis=0)
        o2 = jnp.dot(woT_ref[...], ocm,
                     preferred_element_type=jnp.float32) + wob_ref[...]
        fsum = f_ln + o2

        ones_row = jnp.ones((1, _L), jnp.float32)
        prow = jax.lax.dot_general(ones_row, fsum, (((1,), (1,)), ((), ())),
                                   preferred_element_type=jnp.float32) / _L
        pooled_s[pl.ds(b - 1, 1), :] = prow

    @pl.when((b == _B) & (g == 0))
    def _():
        pooled = pooled_s[...]
        glog = jnp.dot(pooled, gw_ref[...],
                       preferred_element_type=jnp.float32) + gb_ref[...]

        eos = []
        for e in range(_E):
            hh = jnp.dot(pooled, ew1_ref[e],
                         preferred_element_type=jnp.float32) + eb1_ref[e]
            hh = _gelu_exact(hh)
            eo_e = jnp.dot(hh, ew2_ref[e],
                           preferred_element_type=jnp.float32) + eb2_ref[e]
            eos.append(eo_e)
        out_ref[...] = glog
        eo_ref[...] = jnp.concatenate(eos, axis=1)


def _sc_route(glog16, eo16):
    mesh = plsc.VectorSubcoreMesh(core_axis_name="c", subcore_axis_name="s")

    @functools.partial(
        pl.kernel, mesh=mesh,
        out_type=jax.ShapeDtypeStruct((16,), jnp.float32),
        scratch_types=[pltpu.VMEM((16,), jnp.float32),
                       pltpu.VMEM((16,), jnp.float32),
                       pltpu.VMEM((16,), jnp.float32)],
    )
    def _route(glog_hbm, eo_hbm, out_hbm, g_v, e_v, o_v):
        cid = jax.lax.axis_index("c")
        sid = jax.lax.axis_index("s")

        @pl.when((cid == 0) & (sid == 0))
        def _():
            pltpu.sync_copy(glog_hbm, g_v)
            pltpu.sync_copy(eo_hbm, e_v)
            g = g_v[...]
            eo = e_v[...]
            idx = jax.lax.broadcasted_iota(jnp.int32, (16,), 0)

            def bfly(x, op):
                x = op(x, jnp.take(x, idx ^ 1))
                return op(x, jnp.take(x, idx ^ 2))

            m = bfly(g, jnp.maximum)
            ex = jnp.exp(g - m)
            gs = ex / bfly(ex, jnp.add)
            gmin = bfly(gs, jnp.minimum)
            sel = jnp.where(gs <= gmin, idx, -1)
            excl = bfly(sel, jnp.maximum)
            t = jnp.where(idx != excl, gs * eo, jnp.float32(0.0))
            p = bfly(t, jnp.add)
            o_v[...] = jnp.take(p, (idx * 4) & 15)
            pltpu.sync_copy(o_v, out_hbm)

    return _route(glog16, eo16)


def kernel(x, text_features, dc_w, dc_b, conv_w, conv_b, proj, norm1_w,
           norm1_b, norm2_w, norm2_b, wq, wk, wv, wo, wo_b, gate_w, gate_b,
           e_w1, e_b1, e_w2, e_b2):
    B = x.shape[0]
    xr = x.reshape(B, 2 * _NS, _CH, _L)
    dcw = dc_w.reshape(_INC, _INC * 4)
    wtaps = conv_w.transpose(0, 2, 3, 1).reshape(_OUTC, 9 * _INC).astype(
        jnp.bfloat16)

    grid = (B + 1, 2)

    def const(*block):
        return pl.BlockSpec(block, lambda b, g: tuple(0 for _ in block))

    def xspec(j):
        return pl.BlockSpec(
            (1, 1, _CH, _L),
            lambda b, g, j=j: (jnp.minimum(b, _B - 1), g * _NS + j, 0, 0))

    in_specs = [
        xspec(0), xspec(1), xspec(2), xspec(3),
        pl.BlockSpec((1, _T, _INC),
                     lambda b, g: (jnp.maximum(b - 1, 0), 0, 0)),
        const(_INC, _INC * 4),
        const(_INC, 1),
        const(_OUTC, 9 * _INC),
        const(_OUTC, 1),
        const(_INC, _OUTC),
        const(_OUTC, 1), const(_OUTC, 1),
        const(1, _OUTC), const(1, _OUTC),
        const(_OUTC, _OUTC),
        const(_OUTC, _OUTC),
        const(_OUTC, _OUTC),
        const(_OUTC, _OUTC),
        const(_OUTC, 1),
        const(_OUTC, _E),
        const(1, _E),
        const(_E, _OUTC, _OUTC),
        const(_E, 1, _OUTC),
        const(_E, _OUTC, 1),
        const(_E, 1, 1),
    ]

    glog, eo = pl.pallas_call(
        _body,
        grid=grid,
        in_specs=in_specs,
        out_specs=[pl.BlockSpec((_B, _E), lambda b, g: (0, 0)),
                   pl.BlockSpec((_B, _E), lambda b, g: (0, 0))],
        out_shape=[jax.ShapeDtypeStruct((_B, _E), jnp.float32),
                   jax.ShapeDtypeStruct((_B, _E), jnp.float32)],
        scratch_shapes=[
            pltpu.VMEM((2, _INC, _L), jnp.float32),
            pltpu.VMEM((_B, _OUTC), jnp.float32),
        ],
    )(xr, xr, xr, xr, text_features, dcw, dc_b.reshape(_INC, 1), wtaps,
      conv_b.reshape(_OUTC, 1), proj, norm1_w.reshape(_OUTC, 1),
      norm1_b.reshape(_OUTC, 1), norm2_w.reshape(1, _OUTC),
      norm2_b.reshape(1, _OUTC), wq.T, wk, wv, wo.T, wo_b.reshape(_OUTC, 1),
      gate_w, gate_b.reshape(1, _E), e_w1, e_b1.reshape(_E, 1, _OUTC),
      e_w2, e_b2.reshape(_E, 1, 1))

    pred16 = _sc_route(glog.reshape(_B * _E), eo.reshape(_B * _E))
    return pred16[:_B].reshape(_B, 1)
